# baseline (speedup 1.0000x reference)
import functools

import jax
import jax.numpy as jnp
from jax import lax
from jax.experimental import pallas as pl
from jax.experimental.pallas import tpu as pltpu
from jax.experimental.pallas import tpu_sc as plsc

RULE_VOCAB = 100000
TOKEN_VOCAB = 100000
EMB = 128
B, L = 1024, 200
N = B * L

NC, NS = 2, 16
NW = NC * NS
K = N // NW
LANES = 16
VSTEPS = K // LANES
G = 16
NCH_MAX = K // G
PAD = 2 * G
POS_ROWS = NCH_MAX + 2


NB = 8


def _sc_body(typ_hbm, val_hbm, rule_hbm, token_hbm, out_hbm,
             val_v, typ_v, idx0, idx1, pos0, pos1, cnts, *ring):
    bufs = ring[:NB]
    gsems = ring[NB:2 * NB]
    ssems = ring[2 * NB:]
    wid = lax.axis_index("s") * NC + lax.axis_index("c")
    base = wid * K

    pltpu.sync_copy(val_hbm.at[pl.ds(base, K)], val_v)
    pltpu.sync_copy(typ_hbm.at[pl.ds(base, K)], typ_v)

    lane = lax.iota(jnp.int32, LANES)

    lane0 = lane == 0

    @plsc.parallel_loop(0, VSTEPS, 1, unroll=8)
    def _(i):
        t = typ_v[pl.ds(i * LANES, LANES)]
        n0 = plsc.all_reduce_population_count(t == 0)
        plsc.store_scatter(cnts, [jnp.full((LANES,), i, jnp.int32)], n0,
                           mask=lane0)

    def prefix_step(i, carry):
        c = cnts[pl.ds(i * LANES, LANES)]
        s = plsc.cumsum(c)
        cnts[pl.ds(i * LANES, LANES)] = carry + (s - c)
        return carry + jnp.max(s)

    n0 = lax.fori_loop(0, VSTEPS // LANES, prefix_step, jnp.int32(0))
    n1 = jnp.int32(K) - n0

    @plsc.parallel_loop(0, VSTEPS, 1, unroll=8)
    def _(i):
        v = val_v[pl.ds(i * LANES, LANES)]
        t = typ_v[pl.ds(i * LANES, LANES)]
        m0 = t == 0
        c0 = plsc.cumsum(jnp.where(m0, 1, 0))
        o0 = plsc.load_gather(cnts, [jnp.full((LANES,), i, jnp.int32)])
        d0 = o0 + c0 - 1
        d1 = (i * LANES - o0) + (lane + 1 - c0) - 1
        p = base + i * LANES + lane
        m1 = jnp.logical_not(m0)
        plsc.store_scatter(idx0, [d0], v, mask=m0)
        plsc.store_scatter(pos0, [d0], p, mask=m0)
        plsc.store_scatter(idx1, [d1], v, mask=m1)
        plsc.store_scatter(pos1, [d1], p, mask=m1)

    def pad_tail(n_c, idx_b, pos_b):
        @pl.when(n_c > 0)
        def _():
            last = jnp.full((LANES,), n_c - 1, jnp.int32)
            dv = plsc.load_gather(idx_b, [last])
            dp = plsc.load_gather(pos_b, [last])
            for k in range(G // LANES):
                d = n_c + k * LANES + lane
                plsc.store_scatter(idx_b, [d], dv)
                plsc.store_scatter(pos_b, [d], dp)

    pad_tail(n0, idx0, pos0)
    pad_tail(n1, idx1, pos1)

    nch0 = (n0 + (G - 1)) // G
    nch1 = (n1 + (G - 1)) // G
    nct = nch0 + nch1

    def split(j):
        a = (j * nch0) // nct
        a2 = ((j + 1) * nch0) // nct
        is0 = a2 > a
        k = jnp.where(is0, a, j - a)
        c0 = jnp.logical_and(j < nct, is0)
        c1 = jnp.logical_and(j < nct, jnp.logical_not(is0))
        return c0, c1, k

    def gather_desc(j, buf, sem):
        c0, c1, k = split(j)
        return [
            (c0,
             lambda: pltpu.make_async_copy(
                 rule_hbm.at[idx0.at[pl.ds(k * G, G)]], buf, sem)),
            (c1,
             lambda: pltpu.make_async_copy(
                 token_hbm.at[idx1.at[pl.ds(k * G, G)]], buf, sem)),
        ]

    def scatter_desc(j, buf, sem):
        c0, c1, k = split(j)
        return [
            (c0,
             lambda: pltpu.make_async_copy(
                 buf, out_hbm.at[pos0.at[pl.ds(k * G, G)]], sem)),
            (c1,
             lambda: pltpu.make_async_copy(
                 buf, out_hbm.at[pos1.at[pl.ds(k * G, G)]], sem)),
        ]

    def issue(descs):
        for cond, mk in descs:
            @pl.when(cond)
            def _(mk=mk):
                mk().start()

    def drain(descs):
        for cond, mk in descs:
            @pl.when(cond)
            def _(mk=mk):
                mk().wait()

    for b in range(NB):
        issue(gather_desc(jnp.int32(b), bufs[b], gsems[b]))

    def grp(i, _):
        for b in range(NB):
            j = i * NB + b
            drain(gather_desc(j, bufs[b], gsems[b]))
            issue(scatter_desc(j, bufs[b], ssems[b]))
        for b in range(NB):
            j = i * NB + b

            @pl.when(j + NB < nct)
            def _(b=b, j=j):
                drain(scatter_desc(j, bufs[b], ssems[b]))
                issue(gather_desc(j + NB, bufs[b], gsems[b]))
        return 0

    lax.fori_loop(0, (nct + NB - 1) // NB, grp, 0)

    for b in range(NB):
        @pl.when(nct > jnp.int32(b))
        def _(b=b):
            pltpu.make_async_copy(bufs[b], out_hbm.at[pos0.at[pl.ds(0, G)]],
                                  ssems[b]).wait()


@jax.jit
def _action_embed(action_type, action_value, rule_table, token_table):
    typ = action_type.reshape(N).astype(jnp.int32)
    val = action_value.reshape(N).astype(jnp.int32)

    mesh = plsc.VectorSubcoreMesh(core_axis_name="c", subcore_axis_name="s")
    out = pl.kernel(
        _sc_body,
        out_type=jax.ShapeDtypeStruct((N, EMB), jnp.float32),
        mesh=mesh,
        compiler_params=pltpu.CompilerParams(needs_layout_passes=False),
        scratch_types=[
            pltpu.VMEM((K,), jnp.int32),
            pltpu.VMEM((K,), jnp.int32),
            pltpu.VMEM((K + PAD,), jnp.int32),
            pltpu.VMEM((K + PAD,), jnp.int32),
            pltpu.VMEM((K + PAD,), jnp.int32),
            pltpu.VMEM((K + PAD,), jnp.int32),
            pltpu.VMEM((VSTEPS,), jnp.int32),
        ] + [pltpu.VMEM((G, EMB), jnp.float32) for _ in range(NB)]
          + [pltpu.SemaphoreType.DMA for _ in range(2 * NB)],
    )(typ, val, rule_table, token_table)
    return out.reshape(B, L, EMB)


def kernel(action_type, action_value, rule_table, token_table):
    return _action_embed(action_type, action_value, rule_table, token_table)


# speedup vs baseline: 1.0858x; 1.0858x over previous
import jax
import jax.numpy as jnp
from jax import lax
from jax.experimental import pallas as pl
from jax.experimental.pallas import tpu as pltpu
from jax.experimental.pallas import tpu_sc as plsc

RULE_VOCAB = 100000
TOKEN_VOCAB = 100000
EMB = 128
B, L = 1024, 200
N = B * L

NC, NS = 2, 16
NW = NC * NS
K = N // NW
LANES = 16
VSTEPS = K // LANES
G = 32
PAD = 2 * G


NB = 8


def _sc_body(typ_hbm, val_hbm, rule_hbm, token_hbm, out_hbm,
             val_v, typ_v, idx0, idx1, pos0, pos1, cnts, *ring):
    bufs = ring[:NB]
    gsems = ring[NB:2 * NB]
    ssems = ring[2 * NB:]
    wid = lax.axis_index("s") * NC + lax.axis_index("c")
    base = wid * K

    pltpu.sync_copy(val_hbm.at[pl.ds(base, K)], val_v)
    pltpu.sync_copy(typ_hbm.at[pl.ds(base, K)], typ_v)

    lane = lax.iota(jnp.int32, LANES)

    lane0 = lane == 0

    @plsc.parallel_loop(0, VSTEPS, 1, unroll=8)
    def _(i):
        t = typ_v[pl.ds(i * LANES, LANES)]
        n0 = plsc.all_reduce_population_count(t == 0)
        plsc.store_scatter(cnts, [jnp.full((LANES,), i, jnp.int32)], n0,
                           mask=lane0)

    def prefix_step(i, carry):
        c = cnts[pl.ds(i * LANES, LANES)]
        s = plsc.cumsum(c)
        cnts[pl.ds(i * LANES, LANES)] = carry + (s - c)
        return carry + jnp.max(s)

    n0 = lax.fori_loop(0, VSTEPS // LANES, prefix_step, jnp.int32(0))
    n1 = jnp.int32(K) - n0

    @plsc.parallel_loop(0, VSTEPS, 1, unroll=8)
    def _(i):
        v = val_v[pl.ds(i * LANES, LANES)]
        t = typ_v[pl.ds(i * LANES, LANES)]
        m0 = t == 0
        c0 = plsc.cumsum(jnp.where(m0, 1, 0))
        o0 = plsc.load_gather(cnts, [jnp.full((LANES,), i, jnp.int32)])
        d0 = o0 + c0 - 1
        d1 = (i * LANES - o0) + (lane + 1 - c0) - 1
        p = base + i * LANES + lane
        m1 = jnp.logical_not(m0)
        plsc.store_scatter(idx0, [d0], v, mask=m0)
        plsc.store_scatter(pos0, [d0], p, mask=m0)
        plsc.store_scatter(idx1, [d1], v, mask=m1)
        plsc.store_scatter(pos1, [d1], p, mask=m1)

    def pad_tail(n_c, idx_b, pos_b):
        @pl.when(n_c > 0)
        def _():
            last = jnp.full((LANES,), n_c - 1, jnp.int32)
            dv = plsc.load_gather(idx_b, [last])
            dp = plsc.load_gather(pos_b, [last])
            for k in range(G // LANES):
                d = n_c + k * LANES + lane
                plsc.store_scatter(idx_b, [d], dv)
                plsc.store_scatter(pos_b, [d], dp)

    pad_tail(n0, idx0, pos0)
    pad_tail(n1, idx1, pos1)

    nch0 = (n0 + (G - 1)) // G
    nch1 = (n1 + (G - 1)) // G
    nct = nch0 + nch1

    def split(j):
        a = (j * nch0) // nct
        a2 = ((j + 1) * nch0) // nct
        is0 = a2 > a
        k = jnp.where(is0, a, j - a)
        c0 = jnp.logical_and(j < nct, is0)
        c1 = jnp.logical_and(j < nct, jnp.logical_not(is0))
        return c0, c1, k

    def gather_desc(j, buf, sem):
        c0, c1, k = split(j)
        return [
            (c0,
             lambda: pltpu.make_async_copy(
                 rule_hbm.at[idx0.at[pl.ds(k * G, G)]], buf, sem)),
            (c1,
             lambda: pltpu.make_async_copy(
                 token_hbm.at[idx1.at[pl.ds(k * G, G)]], buf, sem)),
        ]

    def scatter_desc(j, buf, sem):
        c0, c1, k = split(j)
        return [
            (c0,
             lambda: pltpu.make_async_copy(
                 buf, out_hbm.at[pos0.at[pl.ds(k * G, G)]], sem)),
            (c1,
             lambda: pltpu.make_async_copy(
                 buf, out_hbm.at[pos1.at[pl.ds(k * G, G)]], sem)),
        ]

    def issue(descs):
        for cond, mk in descs:
            @pl.when(cond)
            def _(mk=mk):
                mk().start()

    def drain(descs):
        for cond, mk in descs:
            @pl.when(cond)
            def _(mk=mk):
                mk().wait()

    for b in range(NB):
        issue(gather_desc(jnp.int32(b), bufs[b], gsems[b]))

    def grp(i, _):
        for b in range(NB):
            j = i * NB + b
            drain(gather_desc(j, bufs[b], gsems[b]))
            issue(scatter_desc(j, bufs[b], ssems[b]))
        for b in range(NB):
            j = i * NB + b

            @pl.when(j + NB < nct)
            def _(b=b, j=j):
                drain(scatter_desc(j, bufs[b], ssems[b]))
                issue(gather_desc(j + NB, bufs[b], gsems[b]))
        return 0

    lax.fori_loop(0, (nct + NB - 1) // NB, grp, 0)

    for b in range(NB):
        @pl.when(nct > jnp.int32(b))
        def _(b=b):
            pltpu.make_async_copy(bufs[b], out_hbm.at[pos0.at[pl.ds(0, G)]],
                                  ssems[b]).wait()


@jax.jit
def _action_embed(action_type, action_value, rule_table, token_table):
    typ = action_type.reshape(N).astype(jnp.int32)
    val = action_value.reshape(N).astype(jnp.int32)

    mesh = plsc.VectorSubcoreMesh(core_axis_name="c", subcore_axis_name="s")
    out = pl.kernel(
        _sc_body,
        out_type=jax.ShapeDtypeStruct((N, EMB), jnp.float32),
        mesh=mesh,
        compiler_params=pltpu.CompilerParams(needs_layout_passes=False),
        scratch_types=[
            pltpu.VMEM((K,), jnp.int32),
            pltpu.VMEM((K,), jnp.int32),
            pltpu.VMEM((K + PAD,), jnp.int32),
            pltpu.VMEM((K + PAD,), jnp.int32),
            pltpu.VMEM((K + PAD,), jnp.int32),
            pltpu.VMEM((K + PAD,), jnp.int32),
            pltpu.VMEM((VSTEPS,), jnp.int32),
        ] + [pltpu.VMEM((G, EMB), jnp.float32) for _ in range(NB)]
          + [pltpu.SemaphoreType.DMA for _ in range(2 * NB)],
    )(typ, val, rule_table, token_table)
    return out.reshape(B, L, EMB)


def kernel(action_type, action_value, rule_table, token_table):
    return _action_embed(action_type, action_value, rule_table, token_table)
